# baseline (speedup 1.0000x reference)
import functools

import jax
import jax.numpy as jnp
from jax import lax
from jax.experimental import pallas as pl
from jax.experimental.pallas import tpu as pltpu
from jax.experimental.pallas import tpu_sc as plsc

N_S = 10000
N_T = 10000
E = 320000
PW = 160
IC = 25
C = 80
NW = 32
EPW = E // NW
NCHUNK = EPW // C
NZCHUNK = N_T // C
UC = 80
NUCHUNK = N_T // UC

f32 = jnp.float32


_DN0 = (((0,), (0,)), ((), ()))


def _prep_body(ea_ref, xs_ref, u_ref, w1aba_ref, b1aa_ref,
               w1abb_ref, b1ab_ref, w1atp_ref, onehot_ref,
               w1bp_ref, w2agg_ref, w2u_ref, b2a_ref,
               ewa_ref, ewb_ref, xw1p_ref, uw_ref, wc_ref):
    ea = ea_ref[...]
    ewa_ref[...] = (jnp.dot(ea, w1aba_ref[...],
                            preferred_element_type=f32)
                    + b1aa_ref[...])
    ewb_ref[...] = (jnp.dot(ea, w1abb_ref[...],
                            preferred_element_type=f32)
                    + b1ab_ref[...])

    @pl.when(pl.program_id(0) == 0)
    def _():
        xw1p_ref[...] = (jnp.dot(xs_ref[...], w1atp_ref[...],
                                 preferred_element_type=f32) + onehot_ref[...])
        uw_ref[...] = (jnp.dot(u_ref[...], w2u_ref[...],
                               preferred_element_type=f32) + b2a_ref[...])
        wc_ref[...] = jnp.dot(w1bp_ref[...], w2agg_ref[...],
                              preferred_element_type=f32)


def _tc_prep(ea, x_s, u, w1aba, b1aa, w1abb, b1ab_c, w1atp, onehot,
             w1bp, w2agg, w2u, b2a_r):
    BE = 3200
    grid = (E // BE,)
    full = lambda a: pl.BlockSpec(a.shape, lambda i: (0,) * a.ndim)
    return pl.pallas_call(
        _prep_body,
        grid=grid,
        in_specs=[
            pl.BlockSpec((BE, 16), lambda i: (i, 0)),
            full(x_s), full(u), full(w1aba), full(b1aa), full(w1abb),
            full(b1ab_c), full(w1atp), full(onehot),
            full(w1bp), full(w2agg), full(w2u), full(b2a_r),
        ],
        out_specs=[
            pl.BlockSpec((BE, 128), lambda i: (i, 0)),
            pl.BlockSpec((BE, 16), lambda i: (i, 0)),
            pl.BlockSpec((N_S, PW), lambda i: (0, 0)),
            pl.BlockSpec((1024, 128), lambda i: (0, 0)),
            pl.BlockSpec((160, 128), lambda i: (0, 0)),
        ],
        out_shape=[
            jax.ShapeDtypeStruct((E, 128), f32),
            jax.ShapeDtypeStruct((E, 16), f32),
            jax.ShapeDtypeStruct((N_S, PW), f32),
            jax.ShapeDtypeStruct((1024, 128), f32),
            jax.ShapeDtypeStruct((160, 128), f32),
        ],
    )(ea, x_s, u, w1aba, b1aa, w1abb, b1ab_c, w1atp, onehot,
      w1bp, w2agg, w2u, b2a_r)


def _sc_body(src_hbm, tgt_hbm, ewa_hbm, ewb_hbm, xw1p_hbm, uw_hbm, bt_hbm,
             aggh_hbm, ug_hbm,
             srcv, tgtv, gbuf, ebufa, ebufb, acc, sem, sema, semb):
    cid = lax.axis_index("c")
    sid = lax.axis_index("s")
    wid = cid * 16 + sid

    def _zrow(r, _):
        for j in range(PW // 16):
            gbuf[r, pl.ds(j * 16, 16)] = jnp.zeros((16,), f32)
        return 0
    lax.fori_loop(0, C, _zrow, 0)
    for k in range(8):
        zc = sid + 16 * k

        @pl.when(zc < NZCHUNK)
        def _():
            pltpu.sync_copy(gbuf, acc.at[pl.ds(zc * C, C)])
    plsc.subcore_barrier()

    ebase = wid * EPW

    def _chunk(i, _):
        off = ebase + i * C
        j = i % IC

        @pl.when(j == 0)
        def _():
            pltpu.sync_copy(src_hbm.at[pl.ds(off, C * IC)], srcv)
            pltpu.sync_copy(tgt_hbm.at[pl.ds(off, C * IC)], tgtv)
        gcp = pltpu.async_copy(xw1p_hbm.at[srcv.at[pl.ds(j * C, C)]],
                               gbuf, sem)
        cpa = pltpu.async_copy(ewa_hbm.at[pl.ds(off, C)], ebufa, sema)
        cpb = pltpu.async_copy(ewb_hbm.at[pl.ds(off, C)], ebufb, semb)
        gcp.wait()
        cpa.wait()
        cpb.wait()

        @plsc.parallel_loop(0, C, 1, unroll=2)
        def _row(r):
            vs = [gbuf[r, pl.ds(j2 * 16, 16)] + ebufa[r, pl.ds(j2 * 16, 16)]
                  for j2 in range(8)]
            vb = gbuf[r, pl.ds(128, 16)] + ebufb[r, pl.ds(0, 16)]
            for j2 in range(8):
                gbuf[r, pl.ds(j2 * 16, 16)] = lax.max(vs[j2], 0.1 * vs[j2])
            gbuf[r, pl.ds(128, 16)] = lax.max(vb, 0.1 * vb)
        pltpu.sync_copy(gbuf, acc.at[tgtv.at[pl.ds(j * C, C)]], add=True)
        return 0
    lax.fori_loop(0, NCHUNK, _chunk, 0)

    plsc.subcore_barrier()

    for k in range(8):
        zc = sid + 16 * k

        @pl.when(zc < NZCHUNK)
        def _():
            row0 = zc * C
            pltpu.sync_copy(acc.at[pl.ds(row0, C)], gbuf)
            pltpu.sync_copy(gbuf, aggh_hbm.at[cid, pl.ds(row0, C)])

    for k in range(4):
        cidx = wid + 32 * k

        @pl.when(cidx < NUCHUNK)
        def _():
            uoff = cidx * UC
            pltpu.sync_copy(bt_hbm.at[pl.ds(uoff, UC)],
                            srcv.at[pl.ds(0, UC)])
            pltpu.async_copy(uw_hbm.at[srcv.at[pl.ds(0, UC)]],
                             ebufa, sema).wait()
            pltpu.sync_copy(ebufa, ug_hbm.at[pl.ds(uoff, UC)])


@functools.partial(
    pl.kernel,
    mesh=plsc.VectorSubcoreMesh(core_axis_name="c", subcore_axis_name="s"),
    compiler_params=pltpu.CompilerParams(use_tc_tiling_on_sc=False),
    out_type=[jax.ShapeDtypeStruct((2, N_T, PW), f32),
              jax.ShapeDtypeStruct((N_T, 128), f32)],
    scratch_types=[
        pltpu.VMEM((C * IC,), jnp.int32),
        pltpu.VMEM((C * IC,), jnp.int32),
        pltpu.VMEM((C, PW), f32),
        pltpu.VMEM((C, 128), f32),
        pltpu.VMEM((C, 16), f32),
        pltpu.VMEM_SHARED((N_T, PW), f32),
        pltpu.SemaphoreType.DMA,
        pltpu.SemaphoreType.DMA,
        pltpu.SemaphoreType.DMA,
    ],
)
def _sc_edge(src, tgt, ewa, ewb, xw1p, uw, batch_t, aggh2, ug,
             srcv, tgtv, gbuf, ebufa, ebufb, acc, sem, sema, semb):
    _sc_body(src, tgt, ewa, ewb, xw1p, uw, batch_t, aggh2, ug,
             srcv, tgtv, gbuf, ebufa, ebufb, acc, sem, sema, semb)


def _out_body(xt_ref, aggh_ref, ug_ref, wxt_ref, wc_ref, w2b_ref,
              b2b_ref, out_ref):
    a = aggh_ref[0] + aggh_ref[1]
    hp = (jnp.dot(xt_ref[...], wxt_ref[...], preferred_element_type=f32)
          + jnp.dot(a, wc_ref[...], preferred_element_type=f32)
          + ug_ref[...])
    h = lax.max(hp, 0.1 * hp)
    out_ref[...] = (jnp.dot(h, w2b_ref[...], preferred_element_type=f32)
                    + b2b_ref[...])


def _tc_out(x_t, aggh2, ug, wxt, wc, w2b, b2b_r):
    BT = 1000
    grid = (N_T // BT,)
    full = lambda a: pl.BlockSpec(a.shape, lambda i: (0,) * a.ndim)
    return pl.pallas_call(
        _out_body,
        grid=grid,
        in_specs=[
            pl.BlockSpec((BT, 128), lambda i: (i, 0)),
            pl.BlockSpec((2, BT, PW), lambda i: (0, i, 0)),
            pl.BlockSpec((BT, 128), lambda i: (i, 0)),
            full(wxt), full(wc), full(w2b), full(b2b_r),
        ],
        out_specs=pl.BlockSpec((BT, 128), lambda i: (i, 0)),
        out_shape=jax.ShapeDtypeStruct((N_T, 128), f32),
    )(x_t, aggh2, ug, wxt, wc, w2b, b2b_r)


def kernel(x_s, x_t, edge_index, edge_attr, u, batch_t,
           W1a, b1a, W1b, b1b, W2a, b2a, W2b, b2b):
    src = edge_index[0]
    tgt = edge_index[1]

    w1atp = jnp.concatenate([W1a[:128], jnp.zeros((128, 16), f32)],
                            axis=1)
    onehot = (jnp.arange(PW) == 144).astype(f32)[None, :]
    w1aba = W1a[128:, :128]
    b1aa = b1a[None, :128]
    w1abb = W1a[128:, 128:]
    b1ab_c = b1a[None, 128:]
    w1bp = jnp.concatenate([W1b, b1b[None, :], jnp.zeros((15, 144), f32)],
                           axis=0)
    w2agg = W2a[128:272]
    wxt = W2a[:128]
    w2u = W2a[272:]
    b2a_r = b2a[None, :]
    b2b_r = b2b[None, :]
    ewa, ewb, xw1p, uw, wc = _tc_prep(edge_attr, x_s, u, w1aba, b1aa,
                                      w1abb, b1ab_c, w1atp, onehot,
                                      w1bp, w2agg, w2u, b2a_r)
    aggh2, ug = _sc_edge(src, tgt, ewa, ewb, xw1p, uw, batch_t)
    return _tc_out(x_t, aggh2, ug, wxt, wc, W2b, b2b_r)


# speedup vs baseline: 1.1541x; 1.1541x over previous
import functools

import jax
import jax.numpy as jnp
from jax import lax
from jax.experimental import pallas as pl
from jax.experimental.pallas import tpu as pltpu
from jax.experimental.pallas import tpu_sc as plsc

N_S = 10000
N_T = 10000
E = 320000
PW = 160
IC = 25
C = 80
NW = 32
EPW = E // NW
NCHUNK = EPW // C
NZCHUNK = N_T // C
UC = 80
NUCHUNK = N_T // UC

f32 = jnp.float32


_DN0 = (((0,), (0,)), ((), ()))


def _prep_body(eat_ref, xs_ref, u_ref, w1aba_ref, b1aa_ref,
               w1abb_ref, b1ab_ref, w1atp_ref, onehot_ref,
               w1bp_ref, w2agg_ref, w2u_ref, b2a_ref,
               ewa_ref, ewb_ref, xw1p_ref, uw_ref, wc_ref):
    eat = eat_ref[...]
    ewa_ref[...] = (lax.dot_general(eat, w1aba_ref[...], _DN0,
                                    preferred_element_type=f32)
                    + b1aa_ref[...])
    ewb_ref[...] = (lax.dot_general(eat, w1abb_ref[...], _DN0,
                                    preferred_element_type=f32)
                    + b1ab_ref[...])

    @pl.when(pl.program_id(0) == 0)
    def _():
        xw1p_ref[...] = (jnp.dot(xs_ref[...], w1atp_ref[...],
                                 preferred_element_type=f32) + onehot_ref[...])
        uw_ref[...] = (jnp.dot(u_ref[...], w2u_ref[...],
                               preferred_element_type=f32) + b2a_ref[...])
        wc_ref[...] = jnp.dot(w1bp_ref[...], w2agg_ref[...],
                              preferred_element_type=f32)


def _tc_prep(eat, x_s, u, w1aba, b1aa, w1abb, b1ab_c, w1atp, onehot,
             w1bp, w2agg, w2u, b2a_r):
    BE = 3200
    grid = (E // BE,)
    full = lambda a: pl.BlockSpec(a.shape, lambda i: (0,) * a.ndim)
    return pl.pallas_call(
        _prep_body,
        grid=grid,
        in_specs=[
            pl.BlockSpec((16, BE), lambda i: (0, i)),
            full(x_s), full(u), full(w1aba), full(b1aa), full(w1abb),
            full(b1ab_c), full(w1atp), full(onehot),
            full(w1bp), full(w2agg), full(w2u), full(b2a_r),
        ],
        out_specs=[
            pl.BlockSpec((BE, 128), lambda i: (i, 0)),
            pl.BlockSpec((BE, 16), lambda i: (i, 0)),
            pl.BlockSpec((N_S, PW), lambda i: (0, 0)),
            pl.BlockSpec((1024, 128), lambda i: (0, 0)),
            pl.BlockSpec((160, 128), lambda i: (0, 0)),
        ],
        out_shape=[
            jax.ShapeDtypeStruct((E, 128), f32),
            jax.ShapeDtypeStruct((E, 16), f32),
            jax.ShapeDtypeStruct((N_S, PW), f32),
            jax.ShapeDtypeStruct((1024, 128), f32),
            jax.ShapeDtypeStruct((160, 128), f32),
        ],
    )(eat, x_s, u, w1aba, b1aa, w1abb, b1ab_c, w1atp, onehot,
      w1bp, w2agg, w2u, b2a_r)


def _sc_body(src_hbm, tgt_hbm, ewa_hbm, ewb_hbm, xw1p_hbm, uw_hbm, bt_hbm,
             aggh_hbm, ug_hbm,
             srcv, tgtv, gbuf, ebufa, ebufb, acc, sem, sema, semb):
    cid = lax.axis_index("c")
    sid = lax.axis_index("s")
    wid = cid * 16 + sid

    def _zrow(r, _):
        for j in range(PW // 16):
            gbuf[r, pl.ds(j * 16, 16)] = jnp.zeros((16,), f32)
        return 0
    lax.fori_loop(0, C, _zrow, 0)
    for k in range(8):
        zc = sid + 16 * k

        @pl.when(zc < NZCHUNK)
        def _():
            pltpu.sync_copy(gbuf, acc.at[pl.ds(zc * C, C)])
    plsc.subcore_barrier()

    ebase = wid * EPW

    def _chunk(i, _):
        off = ebase + i * C
        j = i % IC

        @pl.when(j == 0)
        def _():
            pltpu.sync_copy(src_hbm.at[pl.ds(off, C * IC)], srcv)
            pltpu.sync_copy(tgt_hbm.at[pl.ds(off, C * IC)], tgtv)
        gcp = pltpu.async_copy(xw1p_hbm.at[srcv.at[pl.ds(j * C, C)]],
                               gbuf, sem)
        cpa = pltpu.async_copy(ewa_hbm.at[pl.ds(off, C)], ebufa, sema)
        cpb = pltpu.async_copy(ewb_hbm.at[pl.ds(off, C)], ebufb, semb)
        gcp.wait()
        cpa.wait()
        cpb.wait()

        @plsc.parallel_loop(0, C, 1, unroll=2)
        def _row(r):
            vs = [gbuf[r, pl.ds(j2 * 16, 16)] + ebufa[r, pl.ds(j2 * 16, 16)]
                  for j2 in range(8)]
            vb = gbuf[r, pl.ds(128, 16)] + ebufb[r, pl.ds(0, 16)]
            for j2 in range(8):
                gbuf[r, pl.ds(j2 * 16, 16)] = lax.max(vs[j2], 0.1 * vs[j2])
            gbuf[r, pl.ds(128, 16)] = lax.max(vb, 0.1 * vb)
        pltpu.sync_copy(gbuf, acc.at[tgtv.at[pl.ds(j * C, C)]], add=True)
        return 0
    lax.fori_loop(0, NCHUNK, _chunk, 0)

    plsc.subcore_barrier()

    for k in range(8):
        zc = sid + 16 * k

        @pl.when(zc < NZCHUNK)
        def _():
            row0 = zc * C
            pltpu.sync_copy(acc.at[pl.ds(row0, C)], gbuf)
            pltpu.sync_copy(gbuf, aggh_hbm.at[cid, pl.ds(row0, C)])

    for k in range(4):
        cidx = wid + 32 * k

        @pl.when(cidx < NUCHUNK)
        def _():
            uoff = cidx * UC
            pltpu.sync_copy(bt_hbm.at[pl.ds(uoff, UC)],
                            srcv.at[pl.ds(0, UC)])
            pltpu.async_copy(uw_hbm.at[srcv.at[pl.ds(0, UC)]],
                             ebufa, sema).wait()
            pltpu.sync_copy(ebufa, ug_hbm.at[pl.ds(uoff, UC)])


@functools.partial(
    pl.kernel,
    mesh=plsc.VectorSubcoreMesh(core_axis_name="c", subcore_axis_name="s"),
    compiler_params=pltpu.CompilerParams(use_tc_tiling_on_sc=False),
    out_type=[jax.ShapeDtypeStruct((2, N_T, PW), f32),
              jax.ShapeDtypeStruct((N_T, 128), f32)],
    scratch_types=[
        pltpu.VMEM((C * IC,), jnp.int32),
        pltpu.VMEM((C * IC,), jnp.int32),
        pltpu.VMEM((C, PW), f32),
        pltpu.VMEM((C, 128), f32),
        pltpu.VMEM((C, 16), f32),
        pltpu.VMEM_SHARED((N_T, PW), f32),
        pltpu.SemaphoreType.DMA,
        pltpu.SemaphoreType.DMA,
        pltpu.SemaphoreType.DMA,
    ],
)
def _sc_edge(src, tgt, ewa, ewb, xw1p, uw, batch_t, aggh2, ug,
             srcv, tgtv, gbuf, ebufa, ebufb, acc, sem, sema, semb):
    _sc_body(src, tgt, ewa, ewb, xw1p, uw, batch_t, aggh2, ug,
             srcv, tgtv, gbuf, ebufa, ebufb, acc, sem, sema, semb)


def _out_body(xt_ref, aggh_ref, ug_ref, wxt_ref, wc_ref, w2b_ref,
              b2b_ref, out_ref):
    a = aggh_ref[0] + aggh_ref[1]
    hp = (jnp.dot(xt_ref[...], wxt_ref[...], preferred_element_type=f32)
          + jnp.dot(a, wc_ref[...], preferred_element_type=f32)
          + ug_ref[...])
    h = lax.max(hp, 0.1 * hp)
    out_ref[...] = (jnp.dot(h, w2b_ref[...], preferred_element_type=f32)
                    + b2b_ref[...])


def _tc_out(x_t, aggh2, ug, wxt, wc, w2b, b2b_r):
    BT = 1000
    grid = (N_T // BT,)
    full = lambda a: pl.BlockSpec(a.shape, lambda i: (0,) * a.ndim)
    return pl.pallas_call(
        _out_body,
        grid=grid,
        in_specs=[
            pl.BlockSpec((BT, 128), lambda i: (i, 0)),
            pl.BlockSpec((2, BT, PW), lambda i: (0, i, 0)),
            pl.BlockSpec((BT, 128), lambda i: (i, 0)),
            full(wxt), full(wc), full(w2b), full(b2b_r),
        ],
        out_specs=pl.BlockSpec((BT, 128), lambda i: (i, 0)),
        out_shape=jax.ShapeDtypeStruct((N_T, 128), f32),
    )(x_t, aggh2, ug, wxt, wc, w2b, b2b_r)


def kernel(x_s, x_t, edge_index, edge_attr, u, batch_t,
           W1a, b1a, W1b, b1b, W2a, b2a, W2b, b2b):
    src = edge_index[0]
    tgt = edge_index[1]

    w1atp = jnp.concatenate([W1a[:128], jnp.zeros((128, 16), f32)],
                            axis=1)
    onehot = (jnp.arange(PW) == 144).astype(f32)[None, :]
    w1aba = W1a[128:, :128]
    b1aa = b1a[None, :128]
    w1abb = W1a[128:, 128:]
    b1ab_c = b1a[None, 128:]
    w1bp = jnp.concatenate([W1b, b1b[None, :], jnp.zeros((15, 144), f32)],
                           axis=0)
    w2agg = W2a[128:272]
    wxt = W2a[:128]
    w2u = W2a[272:]
    b2a_r = b2a[None, :]
    b2b_r = b2b[None, :]
    eat = edge_attr.T

    ewa, ewb, xw1p, uw, wc = _tc_prep(eat, x_s, u, w1aba, b1aa,
                                      w1abb, b1ab_c, w1atp, onehot,
                                      w1bp, w2agg, w2u, b2a_r)
    aggh2, ug = _sc_edge(src, tgt, ewa, ewb, xw1p, uw, batch_t)
    return _tc_out(x_t, aggh2, ug, wxt, wc, W2b, b2b_r)


# speedup vs baseline: 1.1670x; 1.0112x over previous
import functools

import jax
import jax.numpy as jnp
from jax import lax
from jax.experimental import pallas as pl
from jax.experimental.pallas import tpu as pltpu
from jax.experimental.pallas import tpu_sc as plsc

N_S = 10000
N_T = 10000
E = 320000
PW = 160
IC = 25
C = 80
NW = 32
EPW = E // NW
NCHUNK = EPW // C
NZCHUNK = N_T // C
UC = 80
NUCHUNK = N_T // UC

f32 = jnp.float32


_DN0 = (((0,), (0,)), ((), ()))


def _prep_body(eat_ref, xs_ref, u_ref, w1aba_ref, b1aa_ref,
               w1abb_ref, b1ab_ref, w1atp_ref, onehot_ref,
               w1bp_ref, w2agg_ref, w2u_ref, b2a_ref,
               ewa_ref, ewb_ref, xw1p_ref, uw_ref, wc_ref):
    eat = eat_ref[...]
    ewa_ref[...] = (lax.dot_general(eat, w1aba_ref[...], _DN0,
                                    preferred_element_type=f32)
                    + b1aa_ref[...])
    ewb_ref[...] = (lax.dot_general(eat, w1abb_ref[...], _DN0,
                                    preferred_element_type=f32)
                    + b1ab_ref[...])

    @pl.when(pl.program_id(0) == 0)
    def _():
        xw1p_ref[...] = (jnp.dot(xs_ref[...], w1atp_ref[...],
                                 preferred_element_type=f32) + onehot_ref[...])
        uw_ref[...] = (jnp.dot(u_ref[...], w2u_ref[...],
                               preferred_element_type=f32) + b2a_ref[...])
        wc_ref[...] = jnp.dot(w1bp_ref[...], w2agg_ref[...],
                              preferred_element_type=f32)


def _tc_prep(eat, x_s, u, w1aba, b1aa, w1abb, b1ab_c, w1atp, onehot,
             w1bp, w2agg, w2u, b2a_r):
    BE = 3200
    grid = (E // BE,)
    full = lambda a: pl.BlockSpec(a.shape, lambda i: (0,) * a.ndim)
    return pl.pallas_call(
        _prep_body,
        grid=grid,
        in_specs=[
            pl.BlockSpec((16, BE), lambda i: (0, i)),
            full(x_s), full(u), full(w1aba), full(b1aa), full(w1abb),
            full(b1ab_c), full(w1atp), full(onehot),
            full(w1bp), full(w2agg), full(w2u), full(b2a_r),
        ],
        out_specs=[
            pl.BlockSpec((BE, 128), lambda i: (i, 0)),
            pl.BlockSpec((BE, 16), lambda i: (i, 0)),
            pl.BlockSpec((N_S, PW), lambda i: (0, 0)),
            pl.BlockSpec((1024, 128), lambda i: (0, 0)),
            pl.BlockSpec((160, 128), lambda i: (0, 0)),
        ],
        out_shape=[
            jax.ShapeDtypeStruct((E, 128), f32),
            jax.ShapeDtypeStruct((E, 16), f32),
            jax.ShapeDtypeStruct((N_S, PW), f32),
            jax.ShapeDtypeStruct((1024, 128), f32),
            jax.ShapeDtypeStruct((160, 128), f32),
        ],
    )(eat, x_s, u, w1aba, b1aa, w1abb, b1ab_c, w1atp, onehot,
      w1bp, w2agg, w2u, b2a_r)


def _sc_body(src_hbm, tgt_hbm, ewa_hbm, ewb_hbm, xw1p_hbm,
             aggh_hbm,
             srcv, tgtv, gbuf, ebufa, ebufb, acc, sem, sema, semb):
    cid = lax.axis_index("c")
    sid = lax.axis_index("s")
    wid = cid * 16 + sid

    def _zrow(r, _):
        for j in range(PW // 16):
            gbuf[r, pl.ds(j * 16, 16)] = jnp.zeros((16,), f32)
        return 0
    lax.fori_loop(0, C, _zrow, 0)
    for k in range(8):
        zc = sid + 16 * k

        @pl.when(zc < NZCHUNK)
        def _():
            pltpu.sync_copy(gbuf, acc.at[pl.ds(zc * C, C)])
    plsc.subcore_barrier()

    ebase = wid * EPW

    def _chunk(i, _):
        off = ebase + i * C
        j = i % IC

        @pl.when(j == 0)
        def _():
            pltpu.sync_copy(src_hbm.at[pl.ds(off, C * IC)], srcv)
            pltpu.sync_copy(tgt_hbm.at[pl.ds(off, C * IC)], tgtv)
        gcp = pltpu.async_copy(xw1p_hbm.at[srcv.at[pl.ds(j * C, C)]],
                               gbuf, sem)
        cpa = pltpu.async_copy(ewa_hbm.at[pl.ds(off, C)], ebufa, sema)
        cpb = pltpu.async_copy(ewb_hbm.at[pl.ds(off, C)], ebufb, semb)
        gcp.wait()
        cpa.wait()
        cpb.wait()

        @plsc.parallel_loop(0, C, 1, unroll=2)
        def _row(r):
            vs = [gbuf[r, pl.ds(j2 * 16, 16)] + ebufa[r, pl.ds(j2 * 16, 16)]
                  for j2 in range(8)]
            vb = gbuf[r, pl.ds(128, 16)] + ebufb[r, pl.ds(0, 16)]
            for j2 in range(8):
                gbuf[r, pl.ds(j2 * 16, 16)] = lax.max(vs[j2], 0.1 * vs[j2])
            gbuf[r, pl.ds(128, 16)] = lax.max(vb, 0.1 * vb)
        pltpu.sync_copy(gbuf, acc.at[tgtv.at[pl.ds(j * C, C)]], add=True)
        return 0
    lax.fori_loop(0, NCHUNK, _chunk, 0)

    plsc.subcore_barrier()

    for k in range(8):
        zc = sid + 16 * k

        @pl.when(zc < NZCHUNK)
        def _():
            row0 = zc * C
            pltpu.sync_copy(acc.at[pl.ds(row0, C)], gbuf)
            pltpu.sync_copy(gbuf, aggh_hbm.at[cid, pl.ds(row0, C)])


@functools.partial(
    pl.kernel,
    mesh=plsc.VectorSubcoreMesh(core_axis_name="c", subcore_axis_name="s"),
    compiler_params=pltpu.CompilerParams(use_tc_tiling_on_sc=False),
    out_type=jax.ShapeDtypeStruct((2, N_T, PW), f32),
    scratch_types=[
        pltpu.VMEM((C * IC,), jnp.int32),
        pltpu.VMEM((C * IC,), jnp.int32),
        pltpu.VMEM((C, PW), f32),
        pltpu.VMEM((C, 128), f32),
        pltpu.VMEM((C, 16), f32),
        pltpu.VMEM_SHARED((N_T, PW), f32),
        pltpu.SemaphoreType.DMA,
        pltpu.SemaphoreType.DMA,
        pltpu.SemaphoreType.DMA,
    ],
)
def _sc_edge(src, tgt, ewa, ewb, xw1p, aggh2,
             srcv, tgtv, gbuf, ebufa, ebufb, acc, sem, sema, semb):
    _sc_body(src, tgt, ewa, ewb, xw1p, aggh2,
             srcv, tgtv, gbuf, ebufa, ebufb, acc, sem, sema, semb)


def _sc_ug_body(uw_hbm, bt_hbm, ug_hbm, uidx, ubuf, sem):
    cid = lax.axis_index("c")
    sid = lax.axis_index("s")
    wid = cid * 16 + sid

    for k in range(4):
        cidx = wid + 32 * k

        @pl.when(cidx < NUCHUNK)
        def _():
            off = cidx * UC
            pltpu.sync_copy(bt_hbm.at[pl.ds(off, UC)], uidx)
            pltpu.async_copy(uw_hbm.at[uidx], ubuf, sem).wait()
            pltpu.sync_copy(ubuf, ug_hbm.at[pl.ds(off, UC)])


@functools.partial(
    pl.kernel,
    mesh=plsc.VectorSubcoreMesh(core_axis_name="c", subcore_axis_name="s"),
    compiler_params=pltpu.CompilerParams(use_tc_tiling_on_sc=False),
    out_type=jax.ShapeDtypeStruct((N_T, 128), f32),
    scratch_types=[
        pltpu.VMEM((UC,), jnp.int32),
        pltpu.VMEM((UC, 128), f32),
        pltpu.SemaphoreType.DMA,
    ],
)
def _sc_ugather(uw, batch_t, ug, uidx, ubuf, sem):
    _sc_ug_body(uw, batch_t, ug, uidx, ubuf, sem)


def _out_body(xt_ref, aggh_ref, ug_ref, wxt_ref, wc_ref, w2b_ref,
              b2b_ref, out_ref):
    a = aggh_ref[0] + aggh_ref[1]
    hp = (jnp.dot(xt_ref[...], wxt_ref[...], preferred_element_type=f32)
          + jnp.dot(a, wc_ref[...], preferred_element_type=f32)
          + ug_ref[...])
    h = lax.max(hp, 0.1 * hp)
    out_ref[...] = (jnp.dot(h, w2b_ref[...], preferred_element_type=f32)
                    + b2b_ref[...])


def _tc_out(x_t, aggh2, ug, wxt, wc, w2b, b2b_r):
    BT = 1000
    grid = (N_T // BT,)
    full = lambda a: pl.BlockSpec(a.shape, lambda i: (0,) * a.ndim)
    return pl.pallas_call(
        _out_body,
        grid=grid,
        in_specs=[
            pl.BlockSpec((BT, 128), lambda i: (i, 0)),
            pl.BlockSpec((2, BT, PW), lambda i: (0, i, 0)),
            pl.BlockSpec((BT, 128), lambda i: (i, 0)),
            full(wxt), full(wc), full(w2b), full(b2b_r),
        ],
        out_specs=pl.BlockSpec((BT, 128), lambda i: (i, 0)),
        out_shape=jax.ShapeDtypeStruct((N_T, 128), f32),
    )(x_t, aggh2, ug, wxt, wc, w2b, b2b_r)


def kernel(x_s, x_t, edge_index, edge_attr, u, batch_t,
           W1a, b1a, W1b, b1b, W2a, b2a, W2b, b2b):
    src = edge_index[0]
    tgt = edge_index[1]

    w1atp = jnp.concatenate([W1a[:128], jnp.zeros((128, 16), f32)],
                            axis=1)
    onehot = (jnp.arange(PW) == 144).astype(f32)[None, :]
    w1aba = W1a[128:, :128]
    b1aa = b1a[None, :128]
    w1abb = W1a[128:, 128:]
    b1ab_c = b1a[None, 128:]
    w1bp = jnp.concatenate([W1b, b1b[None, :], jnp.zeros((15, 144), f32)],
                           axis=0)
    w2agg = W2a[128:272]
    wxt = W2a[:128]
    w2u = W2a[272:]
    b2a_r = b2a[None, :]
    b2b_r = b2b[None, :]
    eat = edge_attr.T

    ewa, ewb, xw1p, uw, wc = _tc_prep(eat, x_s, u, w1aba, b1aa,
                                      w1abb, b1ab_c, w1atp, onehot,
                                      w1bp, w2agg, w2u, b2a_r)
    aggh2 = _sc_edge(src, tgt, ewa, ewb, xw1p)
    ug = _sc_ugather(uw, batch_t)
    return _tc_out(x_t, aggh2, ug, wxt, wc, W2b, b2b_r)
